# baseline (speedup 1.0000x reference)
import jax
import jax.numpy as jnp
from jax.experimental import pallas as pl
from jax.experimental.pallas import tpu as pltpu
from jax.experimental.pallas import tpu_sc as plsc

K = 8192
D = 32
BN = 2048


def _argmax_body(x_ref, embed_ref, iota_ref, codes_ref):
    xb = x_ref[...]
    eb = embed_ref[...]
    dots2 = jax.lax.dot_general(
        (xb + xb).astype(jnp.bfloat16), eb.astype(jnp.bfloat16),
        (((1,), (1,)), ((), ())),
        preferred_element_type=jnp.float32)
    x2 = jnp.sum(xb * xb, axis=1, keepdims=True)
    e2 = jnp.sum(eb * eb, axis=1)[None, :]
    s = (x2 - dots2) + e2
    iota = iota_ref[...]
    big = jnp.float32(3.0e38)
    s0 = s[:, :K // 2]
    s1 = s[:, K // 2:]
    m0 = jnp.min(s0, axis=1, keepdims=True)
    m1 = jnp.min(s1, axis=1, keepdims=True)
    i0 = jnp.min(jnp.where(s0 == m0, iota, big), axis=1).astype(jnp.int32)
    i1 = jnp.min(jnp.where(s1 == m1, iota, big), axis=1).astype(jnp.int32)
    r0 = m0[:, 0].astype(jnp.bfloat16).astype(jnp.float32)
    codes_ref[...] = jnp.where(m1[:, 0] < r0, i1 + K // 2, i0)


def _compute_codes(flat, embed):
    n = flat.shape[0]
    iota = jnp.arange(K // 2, dtype=jnp.float32).reshape(1, K // 2)
    return pl.pallas_call(
        _argmax_body,
        grid=(n // BN,),
        in_specs=[
            pl.BlockSpec((BN, D), lambda i: (i, 0)),
            pl.BlockSpec((K, D), lambda i: (0, 0)),
            pl.BlockSpec((1, K // 2), lambda i: (0, 0)),
        ],
        out_specs=pl.BlockSpec((BN,), lambda i: (i,)),
        out_shape=jax.ShapeDtypeStruct((n,), jnp.int32),
        compiler_params=pltpu.CompilerParams(
            dimension_semantics=("parallel",)),
    )(flat, embed, iota)


def _gather_rows(embed_pad, codes):
    n = codes.shape[0]
    window = 128
    width = embed_pad.shape[1]
    idx = codes.reshape(1, n)
    mesh = plsc.VectorSubcoreMesh(core_axis_name="core",
                                  subcore_axis_name="subcore")

    @pl.kernel(out_type=jax.ShapeDtypeStruct((n, width), embed_pad.dtype),
               mesh=mesh)
    def gather_kernel(embed_hbm, idx_hbm, out_hbm):
        def body(i_vmem, o_vmem):
            pltpu.sync_copy(embed_hbm.at[i_vmem.at[0]], o_vmem)

        pltpu.emit_pipeline(
            body,
            grid=(n // window,),
            in_specs=[pl.BlockSpec((1, window), index_map=lambda i: (0, i))],
            out_specs=[pl.BlockSpec((window, width), index_map=lambda i: (i, 0))],
            core_axis_name=("core", "subcore"),
            dimension_semantics=(pltpu.PARALLEL,),
        )(idx_hbm, out_hbm)

    return gather_kernel(embed_pad, idx)


def kernel(x, embed):
    shape = x.shape
    flat = x.reshape(-1, shape[-1])
    n = flat.shape[0]
    h = n // 2
    embed_pad = jnp.pad(embed, ((0, 0), (0, 128 - D)))
    c_a = _compute_codes(flat[:h], embed)
    g_a = _gather_rows(embed_pad, c_a)
    c_b = _compute_codes(flat[h:], embed)
    g_b = _gather_rows(embed_pad, c_b)
    quantized = jnp.concatenate([g_a[:, :D], g_b[:, :D]], axis=0).reshape(shape)
    codes = jnp.concatenate([c_a, c_b], axis=0).reshape(shape[:-1])
    return (quantized, codes)


# speedup vs baseline: 1.0606x; 1.0606x over previous
import jax
import jax.numpy as jnp
from jax.experimental import pallas as pl
from jax.experimental.pallas import tpu as pltpu
from jax.experimental.pallas import tpu_sc as plsc

K = 8192
D = 32
BN = 2048


def _argmax_body(x_ref, embed_ref, iota_ref, codes_ref):
    xb = x_ref[...]
    eb = embed_ref[...]
    dots2 = jax.lax.dot_general(
        (xb + xb).astype(jnp.bfloat16), eb.astype(jnp.bfloat16),
        (((1,), (1,)), ((), ())),
        preferred_element_type=jnp.float32)
    x2 = jnp.sum(xb * xb, axis=1, keepdims=True)
    e2 = jnp.sum(eb * eb, axis=1)[None, :]
    s = (x2 - dots2) + e2
    iota = iota_ref[...]
    big = jnp.float32(3.0e38)
    s0 = s[:, :K // 2]
    s1 = s[:, K // 2:]
    m0 = jnp.min(s0, axis=1, keepdims=True)
    m1 = jnp.min(s1, axis=1, keepdims=True)
    i0 = jnp.min(jnp.where(s0 == m0, iota, big), axis=1).astype(jnp.int32)
    i1 = jnp.min(jnp.where(s1 == m1, iota, big), axis=1).astype(jnp.int32)
    r0 = m0[:, 0].astype(jnp.bfloat16).astype(jnp.float32)
    codes_ref[...] = jnp.where(m1[:, 0] < r0, i1 + K // 2, i0)


def _compute_codes(flat, embed):
    n = flat.shape[0]
    iota = jnp.arange(K // 2, dtype=jnp.float32).reshape(1, K // 2)
    return pl.pallas_call(
        _argmax_body,
        grid=(n // BN,),
        in_specs=[
            pl.BlockSpec((BN, D), lambda i: (i, 0)),
            pl.BlockSpec((K, D), lambda i: (0, 0)),
            pl.BlockSpec((1, K // 2), lambda i: (0, 0)),
        ],
        out_specs=pl.BlockSpec((BN,), lambda i: (i,)),
        out_shape=jax.ShapeDtypeStruct((n,), jnp.int32),
        compiler_params=pltpu.CompilerParams(
            dimension_semantics=("parallel",)),
    )(flat, embed, iota)


def _gather_rows(embed_pad, codes):
    n = codes.shape[0]
    window = 128
    width = embed_pad.shape[1]
    idx = codes.reshape(1, n)
    mesh = plsc.VectorSubcoreMesh(core_axis_name="core",
                                  subcore_axis_name="subcore")

    @pl.kernel(out_type=jax.ShapeDtypeStruct((n, width), embed_pad.dtype),
               mesh=mesh)
    def gather_kernel(embed_hbm, idx_hbm, out_hbm):
        def body(i_vmem, o_vmem):
            pltpu.sync_copy(embed_hbm.at[i_vmem.at[0]], o_vmem)

        pltpu.emit_pipeline(
            body,
            grid=(n // window,),
            in_specs=[pl.BlockSpec((1, window), index_map=lambda i: (0, i))],
            out_specs=[pl.BlockSpec((window, width), index_map=lambda i: (i, 0))],
            core_axis_name=("core", "subcore"),
            dimension_semantics=(pltpu.PARALLEL,),
        )(idx_hbm, out_hbm)

    return gather_kernel(embed_pad, idx)


def kernel(x, embed):
    shape = x.shape
    flat = x.reshape(-1, shape[-1])
    codes_flat = _compute_codes(flat, embed)
    embed_pad = jnp.pad(embed, ((0, 0), (0, 128 - D)))
    quantized = _gather_rows(embed_pad, codes_flat)[:, :D].reshape(shape)
    codes = codes_flat.reshape(shape[:-1])
    return (quantized, codes)
